# baseline (speedup 1.0000x reference)
import functools

import jax
import jax.numpy as jnp
from jax import lax
from jax.experimental import pallas as pl
from jax.experimental.pallas import tpu as pltpu
from jax.experimental.pallas import tpu_sc as plsc

N_NODES = 10000
N_PAD = 10240
E_EDGES = 160000
E_PAD = 163840
D_IN = 256
D_H = 512
NC, NS = 2, 16

_MESH = plsc.VectorSubcoreMesh(core_axis_name="c", subcore_axis_name="s")


def _make_agg(C):
    Cp = C // 2
    EB = 128
    ept = E_PAD // NS
    n_iter = ept // EB
    rpt = N_PAD // NS

    @functools.partial(
        pl.kernel,
        out_type=jax.ShapeDtypeStruct((C, N_PAD, 128), jnp.float32),
        mesh=_MESH,
        compiler_params=pltpu.CompilerParams(use_tc_tiling_on_sc=False, needs_layout_passes=False),
        scratch_types=[
            pltpu.VMEM((EB,), jnp.int32),
            pltpu.VMEM((EB,), jnp.int32),
            pltpu.VMEM((EB, 128), jnp.float32),
            pltpu.VMEM_SHARED((N_PAD, 128), jnp.float32),
            pltpu.SemaphoreType.DMA,
        ],
    )
    def agg_kernel(h_hbm, src_hbm, dst_hbm, out_hbm, src_v, dst_v, rows_v,
                   acc_s, sem):
        cid = lax.axis_index("c")
        sid = lax.axis_index("s")
        ebase = sid * ept
        rbase = sid * rpt

        for c in range(C):
            @pl.when(cid == c // Cp)
            def _():
                def zr(i, carry):
                    rows_v[i // 8, pl.ds((i % 8) * 16, 16)] = jnp.zeros(
                        (16,), jnp.float32)
                    return carry
                lax.fori_loop(0, EB * 8, zr, 0)
                for k in range(rpt // EB):
                    pltpu.sync_copy(rows_v,
                                    acc_s.at[pl.ds(rbase + k * EB, EB)])
                plsc.subcore_barrier()

                def body(g, carry):
                    off = ebase + g * EB
                    pltpu.sync_copy(src_hbm.at[pl.ds(off, EB)], src_v)
                    pltpu.sync_copy(dst_hbm.at[pl.ds(off, EB)], dst_v)
                    pltpu.async_copy(h_hbm.at[c].at[src_v], rows_v,
                                     sem).wait()
                    pltpu.sync_copy(rows_v, acc_s.at[dst_v], add=True)
                    return carry
                lax.fori_loop(0, n_iter, body, 0)
                plsc.subcore_barrier()

                for k in range(rpt // EB):
                    pltpu.sync_copy(acc_s.at[pl.ds(rbase + k * EB, EB)],
                                    rows_v)
                    pltpu.sync_copy(
                        rows_v, out_hbm.at[c].at[pl.ds(rbase + k * EB, EB)])

    return agg_kernel


def _make_cnt():
    EB = 128
    epc = E_PAD // NC
    ept = epc // NS
    n_iter = ept // EB
    rpt = N_PAD // NS

    @functools.partial(
        pl.kernel,
        out_type=jax.ShapeDtypeStruct((NC, N_PAD, 128), jnp.float32),
        mesh=_MESH,
        compiler_params=pltpu.CompilerParams(use_tc_tiling_on_sc=False, needs_layout_passes=False),
        scratch_types=[
            pltpu.VMEM((EB,), jnp.int32),
            pltpu.VMEM((EB, 128), jnp.float32),
            pltpu.VMEM_SHARED((N_PAD, 128), jnp.float32),
        ],
    )
    def cnt_kernel(dst_hbm, out_hbm, dst_v, ones_v, acc_s):
        cid = lax.axis_index("c")
        sid = lax.axis_index("s")
        ebase = cid * epc + sid * ept
        rbase = sid * rpt

        def fill(val):
            def f(i, carry):
                ones_v[i // 8, pl.ds((i % 8) * 16, 16)] = jnp.full(
                    (16,), val, jnp.float32)
                return carry
            lax.fori_loop(0, EB * 8, f, 0)

        fill(0.0)
        for k in range(rpt // EB):
            pltpu.sync_copy(ones_v, acc_s.at[pl.ds(rbase + k * EB, EB)])
        plsc.subcore_barrier()
        fill(1.0)

        def body(g, carry):
            off = ebase + g * EB
            pltpu.sync_copy(dst_hbm.at[pl.ds(off, EB)], dst_v)
            pltpu.sync_copy(ones_v, acc_s.at[dst_v], add=True)
            return carry
        lax.fori_loop(0, n_iter, body, 0)
        plsc.subcore_barrier()

        for k in range(rpt // EB):
            pltpu.sync_copy(acc_s.at[pl.ds(rbase + k * EB, EB)], ones_v)
            pltpu.sync_copy(ones_v,
                            out_hbm.at[cid].at[pl.ds(rbase + k * EB, EB)])

    return cnt_kernel


def _make_tc_layer(C):
    D = C * 128
    BN = 512
    grid = (N_PAD // BN,)

    def body(h_ref, agg_ref, cnt_ref, wl_ref, wr_ref, b_ref, out_ref):
        cnt = cnt_ref[0] + cnt_ref[1]
        inv = 1.0 / jnp.maximum(cnt, 1.0)
        s = jnp.zeros((BN, D_H), jnp.float32)
        for c in range(C):
            mean_c = agg_ref[c] * inv
            s += jnp.dot(mean_c, wl_ref[pl.ds(c * 128, 128), :],
                         preferred_element_type=jnp.float32)
            s += jnp.dot(h_ref[c], wr_ref[pl.ds(c * 128, 128), :],
                         preferred_element_type=jnp.float32)
        s = jnp.maximum(s + b_ref[0][None, :], 0.0)
        for c2 in range(D_H // 128):
            out_ref[c2] = s[:, c2 * 128:(c2 + 1) * 128]

    return pl.pallas_call(
        body,
        grid=grid,
        in_specs=[
            pl.BlockSpec((C, BN, 128), lambda i: (0, i, 0)),
            pl.BlockSpec((C, BN, 128), lambda i: (0, i, 0)),
            pl.BlockSpec((NC, BN, 128), lambda i: (0, i, 0)),
            pl.BlockSpec((D, D_H), lambda i: (0, 0)),
            pl.BlockSpec((D, D_H), lambda i: (0, 0)),
            pl.BlockSpec((1, D_H), lambda i: (0, 0)),
        ],
        out_specs=pl.BlockSpec((D_H // 128, BN, 128), lambda i: (0, i, 0)),
        out_shape=jax.ShapeDtypeStruct((D_H // 128, N_PAD, 128), jnp.float32),
    )


def _make_tc_final():
    C = D_H // 128
    BN = 512
    grid = (N_PAD // BN,)

    def body(h_ref, w_ref, b_ref, out_ref):
        s = jnp.zeros((BN, D_H), jnp.float32)
        for c in range(C):
            s += jnp.dot(h_ref[c], w_ref[pl.ds(c * 128, 128), :],
                         preferred_element_type=jnp.float32)
        out_ref[...] = s + b_ref[0][None, :]

    return pl.pallas_call(
        body,
        grid=grid,
        in_specs=[
            pl.BlockSpec((C, BN, 128), lambda i: (0, i, 0)),
            pl.BlockSpec((D_H, D_H), lambda i: (0, 0)),
            pl.BlockSpec((1, D_H), lambda i: (0, 0)),
        ],
        out_specs=pl.BlockSpec((BN, D_H), lambda i: (i, 0)),
        out_shape=jax.ShapeDtypeStruct((N_PAD, D_H), jnp.float32),
    )


def _make_decoder():
    EB = 64
    epw = E_PAD // (NC * NS)
    n_iter = epw // EB
    KV = D_H // 16

    @functools.partial(
        pl.kernel,
        out_type=jax.ShapeDtypeStruct((E_PAD,), jnp.float32),
        mesh=_MESH,
        compiler_params=pltpu.CompilerParams(use_tc_tiling_on_sc=False, needs_layout_passes=False),
        scratch_types=[
            pltpu.VMEM((EB,), jnp.int32),
            pltpu.VMEM((EB,), jnp.int32),
            pltpu.VMEM((EB, D_H), jnp.float32),
            pltpu.VMEM((EB, D_H), jnp.float32),
            pltpu.VMEM((EB,), jnp.float32),
            pltpu.SemaphoreType.DMA,
        ],
    )
    def dec_kernel(za_hbm, zb_hbm, ea_hbm, eb_hbm, out_hbm,
                   ia_v, ib_v, ra_v, rb_v, o_v, sem):
        cid = lax.axis_index("c")
        sid = lax.axis_index("s")
        base = (sid * NC + cid) * epw

        lane = lax.iota(jnp.int32, 16)

        def body(g, carry):
            off = base + g * EB
            pltpu.sync_copy(ea_hbm.at[pl.ds(off, EB)], ia_v)
            pltpu.sync_copy(eb_hbm.at[pl.ds(off, EB)], ib_v)
            pltpu.async_copy(za_hbm.at[ia_v], ra_v, sem).wait()
            pltpu.async_copy(zb_hbm.at[ib_v], rb_v, sem).wait()

            def grp(g2, carry2):
                rows = g2 * 16 + lane

                def kblk(kb, acc):
                    for kk in range(16):
                        col = kb * 16 + kk
                        colv = jnp.full((16,), col, jnp.int32)
                        va = plsc.load_gather(ra_v, [rows, colv])
                        vb = plsc.load_gather(rb_v, [rows, colv])
                        acc = acc + va * vb
                    return acc
                acc = lax.fori_loop(0, KV, kblk, jnp.zeros((16,),
                                                           jnp.float32))
                o_v[pl.ds(pl.multiple_of(g2 * 16, 16), 16)] = (
                    1.0 / (1.0 + jnp.exp(-acc)))
                return carry2
            lax.fori_loop(0, EB // 16, grp, 0)

            pltpu.sync_copy(o_v, out_hbm.at[pl.ds(off, EB)])
            return carry
        lax.fori_loop(0, n_iter, body, 0)

    return dec_kernel


_AGG = {2: _make_agg(2), 4: _make_agg(4)}
_CNT = _make_cnt()
_TCL = {2: _make_tc_layer(2), 4: _make_tc_layer(4)}
_TCF = _make_tc_final()
_DEC = _make_decoder()


def _chunk(z):
    n, d = z.shape
    zp = jnp.pad(z, ((0, N_PAD - n), (0, 0)))
    return jnp.transpose(zp.reshape(N_PAD, d // 128, 128), (1, 0, 2))


def kernel(z1, z2, edge_index, backbones, Wl0, Wr0, b0, Wl1, Wr1, b1,
           Wl2, Wr2, b2, Wlin, blin):
    pad_e = E_PAD - E_EDGES
    src = jnp.pad(backbones[0], (0, pad_e), constant_values=N_NODES)
    dst = jnp.pad(backbones[1], (0, pad_e), constant_values=N_NODES)
    ea = jnp.pad(edge_index[0], (0, pad_e))
    eb = jnp.pad(edge_index[1], (0, pad_e))

    cnt = _CNT(dst)

    params = [(Wl0, Wr0, b0), (Wl1, Wr1, b1), (Wl2, Wr2, b2)]

    def tower(z):
        h = _chunk(z)
        for (Wl, Wr, b) in params:
            C = h.shape[0]
            agg = _AGG[C](h, src, dst)
            h = _TCL[C](h, agg, cnt, Wl, Wr, b.reshape(1, D_H))
        return _TCF(h, Wlin, blin.reshape(1, D_H))

    za = tower(z1)
    zb = tower(z2)
    return _DEC(za, zb, ea, eb)[:E_EDGES]


# speedup vs baseline: 1.2960x; 1.2960x over previous
import functools

import jax
import jax.numpy as jnp
from jax import lax
from jax.experimental import pallas as pl
from jax.experimental.pallas import tpu as pltpu
from jax.experimental.pallas import tpu_sc as plsc

N_NODES = 10000
N_PAD = 10240
E_EDGES = 160000
E_PAD = 163840
D_IN = 256
D_H = 512
NC, NS = 2, 16

_MESH = plsc.VectorSubcoreMesh(core_axis_name="c", subcore_axis_name="s")


def _make_agg(C):
    Cp = C // 2
    EB = 128
    ept = E_PAD // NS
    n_iter = ept // EB
    rpt = N_PAD // NS

    @functools.partial(
        pl.kernel,
        out_type=jax.ShapeDtypeStruct((C, N_PAD, 128), jnp.float32),
        mesh=_MESH,
        compiler_params=pltpu.CompilerParams(use_tc_tiling_on_sc=False, needs_layout_passes=False),
        scratch_types=[
            pltpu.VMEM((2, EB), jnp.int32),
            pltpu.VMEM((2, EB), jnp.int32),
            pltpu.VMEM((2, EB, 128), jnp.float32),
            pltpu.VMEM_SHARED((N_PAD, 128), jnp.float32),
            pltpu.SemaphoreType.DMA,
            pltpu.SemaphoreType.DMA,
            pltpu.SemaphoreType.DMA,
            pltpu.SemaphoreType.DMA,
        ],
    )
    def agg_kernel(h_hbm, src_hbm, dst_hbm, out_hbm, src_v, dst_v, rows_v,
                   acc_s, sg0, sg1, si0, si1):
        cid = lax.axis_index("c")
        sid = lax.axis_index("s")
        ebase = sid * ept
        rbase = sid * rpt
        sgs = (sg0, sg1)
        sis = (si0, si1)

        def idx_desc(g, slot):
            off = ebase + (g % n_iter) * EB
            return (pltpu.make_async_copy(src_hbm.at[pl.ds(off, EB)],
                                          src_v.at[slot], sis[slot]),
                    pltpu.make_async_copy(dst_hbm.at[pl.ds(off, EB)],
                                          dst_v.at[slot], sis[slot]))

        for c in range(C):
            @pl.when(cid == c // Cp)
            def _():
                def zr(i, carry):
                    rows_v[0, i // 8, pl.ds((i % 8) * 16, 16)] = jnp.zeros(
                        (16,), jnp.float32)
                    return carry
                lax.fori_loop(0, EB * 8, zr, 0)
                for k in range(rpt // EB):
                    pltpu.sync_copy(rows_v.at[0],
                                    acc_s.at[pl.ds(rbase + k * EB, EB)])
                plsc.subcore_barrier()

                def gather(slot):
                    return pltpu.make_async_copy(
                        h_hbm.at[c].at[src_v.at[slot]], rows_v.at[slot],
                        sgs[slot])

                def load_idx(g, slot):
                    for d in idx_desc(g, slot):
                        d.start()

                def wait_idx(g, slot):
                    for d in idx_desc(g, slot):
                        d.wait()

                def step(g, s, do_idx, do_gather):
                    gather(s).wait()
                    if do_gather:
                        wait_idx(g + 1, 1 - s)
                        gather(1 - s).start()
                    pltpu.sync_copy(rows_v.at[s], acc_s.at[dst_v.at[s]],
                                    add=True)
                    if do_idx:
                        load_idx(g + 2, s)

                load_idx(0, 0)
                wait_idx(0, 0)
                gather(0).start()
                load_idx(1, 1)
                step(0, 0, True, True)

                def body(t, carry):
                    step(1 + 2 * t, 1, True, True)
                    step(2 + 2 * t, 0, True, True)
                    return carry
                lax.fori_loop(0, (n_iter - 4) // 2, body, 0)
                step(n_iter - 3, 1, True, True)
                step(n_iter - 2, 0, False, True)
                step(n_iter - 1, 1, False, False)
                plsc.subcore_barrier()

                for k in range(rpt // EB):
                    pltpu.sync_copy(acc_s.at[pl.ds(rbase + k * EB, EB)],
                                    rows_v.at[0])
                    pltpu.sync_copy(
                        rows_v.at[0],
                        out_hbm.at[c].at[pl.ds(rbase + k * EB, EB)])

    return agg_kernel


def _make_cnt():
    EB = 128
    epc = E_PAD // NC
    ept = epc // NS
    n_iter = ept // EB
    rpt = N_PAD // NS

    @functools.partial(
        pl.kernel,
        out_type=jax.ShapeDtypeStruct((NC, N_PAD, 128), jnp.float32),
        mesh=_MESH,
        compiler_params=pltpu.CompilerParams(use_tc_tiling_on_sc=False, needs_layout_passes=False),
        scratch_types=[
            pltpu.VMEM((n_iter, EB), jnp.int32),
            pltpu.VMEM((EB, 128), jnp.float32),
            pltpu.VMEM_SHARED((N_PAD, 128), jnp.float32),
        ],
    )
    def cnt_kernel(dst_hbm, out_hbm, dst_all, ones_v, acc_s):
        cid = lax.axis_index("c")
        sid = lax.axis_index("s")
        row0 = cid * (epc // EB) + sid * n_iter
        rbase = sid * rpt

        pltpu.sync_copy(dst_hbm.at[pl.ds(row0, n_iter)], dst_all)

        def fill(val):
            def f(i, carry):
                ones_v[i // 8, pl.ds((i % 8) * 16, 16)] = jnp.full(
                    (16,), val, jnp.float32)
                return carry
            lax.fori_loop(0, EB * 8, f, 0)

        fill(0.0)
        for k in range(rpt // EB):
            pltpu.sync_copy(ones_v, acc_s.at[pl.ds(rbase + k * EB, EB)])
        plsc.subcore_barrier()
        fill(1.0)

        def body(g, carry):
            pltpu.sync_copy(ones_v, acc_s.at[dst_all.at[g]], add=True)
            return carry
        lax.fori_loop(0, n_iter, body, 0)
        plsc.subcore_barrier()

        for k in range(rpt // EB):
            pltpu.sync_copy(acc_s.at[pl.ds(rbase + k * EB, EB)], ones_v)
            pltpu.sync_copy(ones_v,
                            out_hbm.at[cid].at[pl.ds(rbase + k * EB, EB)])

    return cnt_kernel


def _make_tc_layer(C):
    D = C * 128
    BN = 512
    grid = (N_PAD // BN,)

    def body(h_ref, agg_ref, cnt_ref, wl_ref, wr_ref, b_ref, out_ref):
        cnt = cnt_ref[0] + cnt_ref[1]
        inv = 1.0 / jnp.maximum(cnt, 1.0)
        s = jnp.zeros((BN, D_H), jnp.float32)
        for c in range(C):
            mean_c = agg_ref[c] * inv
            s += jnp.dot(mean_c, wl_ref[pl.ds(c * 128, 128), :],
                         preferred_element_type=jnp.float32)
            s += jnp.dot(h_ref[c], wr_ref[pl.ds(c * 128, 128), :],
                         preferred_element_type=jnp.float32)
        s = jnp.maximum(s + b_ref[0][None, :], 0.0)
        for c2 in range(D_H // 128):
            out_ref[c2] = s[:, c2 * 128:(c2 + 1) * 128]

    return pl.pallas_call(
        body,
        grid=grid,
        in_specs=[
            pl.BlockSpec((C, BN, 128), lambda i: (0, i, 0)),
            pl.BlockSpec((C, BN, 128), lambda i: (0, i, 0)),
            pl.BlockSpec((NC, BN, 128), lambda i: (0, i, 0)),
            pl.BlockSpec((D, D_H), lambda i: (0, 0)),
            pl.BlockSpec((D, D_H), lambda i: (0, 0)),
            pl.BlockSpec((1, D_H), lambda i: (0, 0)),
        ],
        out_specs=pl.BlockSpec((D_H // 128, BN, 128), lambda i: (0, i, 0)),
        out_shape=jax.ShapeDtypeStruct((D_H // 128, N_PAD, 128), jnp.float32),
    )


def _make_tc_final():
    C = D_H // 128
    BN = 512
    grid = (N_PAD // BN,)

    def body(h_ref, w_ref, b_ref, out_ref):
        s = jnp.zeros((BN, D_H), jnp.float32)
        for c in range(C):
            s += jnp.dot(h_ref[c], w_ref[pl.ds(c * 128, 128), :],
                         preferred_element_type=jnp.float32)
        out_ref[...] = s + b_ref[0][None, :]

    return pl.pallas_call(
        body,
        grid=grid,
        in_specs=[
            pl.BlockSpec((C, BN, 128), lambda i: (0, i, 0)),
            pl.BlockSpec((D_H, D_H), lambda i: (0, 0)),
            pl.BlockSpec((1, D_H), lambda i: (0, 0)),
        ],
        out_specs=pl.BlockSpec((BN, D_H), lambda i: (i, 0)),
        out_shape=jax.ShapeDtypeStruct((N_PAD, D_H), jnp.float32),
    )


def _make_decoder():
    EB = 32
    epw = E_PAD // (NC * NS)
    n_iter = epw // EB
    KV = D_H // 16

    @functools.partial(
        pl.kernel,
        out_type=jax.ShapeDtypeStruct((E_PAD,), jnp.float32),
        mesh=_MESH,
        compiler_params=pltpu.CompilerParams(use_tc_tiling_on_sc=False, needs_layout_passes=False),
        scratch_types=[
            pltpu.VMEM((n_iter, EB), jnp.int32),
            pltpu.VMEM((n_iter, EB), jnp.int32),
            pltpu.VMEM((2, EB, D_H), jnp.float32),
            pltpu.VMEM((2, EB, D_H), jnp.float32),
            pltpu.VMEM((epw,), jnp.float32),
            pltpu.SemaphoreType.DMA,
            pltpu.SemaphoreType.DMA,
            pltpu.SemaphoreType.DMA,
            pltpu.SemaphoreType.DMA,
        ],
    )
    def dec_kernel(za_hbm, zb_hbm, ea_hbm, eb_hbm, out_hbm,
                   ia_all, ib_all, ra_v, rb_v, o_all, sa0, sa1, sb0, sb1):
        cid = lax.axis_index("c")
        sid = lax.axis_index("s")
        wid = sid * NC + cid
        sas = (sa0, sa1)
        sbs = (sb0, sb1)

        pltpu.sync_copy(ea_hbm.at[pl.ds(wid * n_iter, n_iter)], ia_all)
        pltpu.sync_copy(eb_hbm.at[pl.ds(wid * n_iter, n_iter)], ib_all)

        lane = lax.iota(jnp.int32, 16)

        def start(g, slot):
            pltpu.async_copy(za_hbm.at[ia_all.at[g]], ra_v.at[slot],
                             sas[slot])
            pltpu.async_copy(zb_hbm.at[ib_all.at[g]], rb_v.at[slot],
                             sbs[slot])

        def wait(g, slot):
            pltpu.make_async_copy(za_hbm.at[ia_all.at[g]], ra_v.at[slot],
                                  sas[slot]).wait()
            pltpu.make_async_copy(zb_hbm.at[ib_all.at[g]], rb_v.at[slot],
                                  sbs[slot]).wait()

        def compute(g, s):
            for g2 in range(EB // 16):
                rows = g2 * 16 + lane

                def kblk(kb, acc):
                    for kk in range(16):
                        col = kb * 16 + kk
                        colv = jnp.full((16,), col, jnp.int32)
                        va = plsc.load_gather(ra_v.at[s], [rows, colv])
                        vb = plsc.load_gather(rb_v.at[s], [rows, colv])
                        acc = acc + va * vb
                    return acc
                acc = lax.fori_loop(0, KV, kblk,
                                    jnp.zeros((16,), jnp.float32))
                o_all[pl.ds(g * EB + g2 * 16, 16)] = (
                    1.0 / (1.0 + jnp.exp(-acc)))

        start(0, 0)
        start(1, 1)

        def body(t, carry):
            for s in range(2):
                g = 2 * t + s
                wait(g, s)
                start(g + 2, s)
                compute(g, s)
            return carry
        lax.fori_loop(0, n_iter // 2 - 1, body, 0)
        for s in range(2):
            g = n_iter - 2 + s
            wait(g, s)
            compute(g, s)

        pltpu.sync_copy(o_all, out_hbm.at[pl.ds(wid * epw, epw)])

    return dec_kernel


_AGG = {2: _make_agg(2), 4: _make_agg(4)}
_CNT = _make_cnt()
_TCL = {2: _make_tc_layer(2), 4: _make_tc_layer(4)}
_TCF = _make_tc_final()
_DEC = _make_decoder()


def _chunk(z):
    n, d = z.shape
    zp = jnp.pad(z, ((0, N_PAD - n), (0, 0)))
    return jnp.transpose(zp.reshape(N_PAD, d // 128, 128), (1, 0, 2))


def kernel(z1, z2, edge_index, backbones, Wl0, Wr0, b0, Wl1, Wr1, b1,
           Wl2, Wr2, b2, Wlin, blin):
    pad_e = E_PAD - E_EDGES
    src = jnp.pad(backbones[0], (0, pad_e), constant_values=N_NODES)
    dst = jnp.pad(backbones[1], (0, pad_e), constant_values=N_NODES)
    ea = jnp.pad(edge_index[0], (0, pad_e)).reshape(E_PAD // 32, 32)
    eb = jnp.pad(edge_index[1], (0, pad_e)).reshape(E_PAD // 32, 32)

    cnt = _CNT(dst.reshape(E_PAD // 128, 128))

    params = [(Wl0, Wr0, b0), (Wl1, Wr1, b1), (Wl2, Wr2, b2)]

    def tower(z):
        h = _chunk(z)
        for (Wl, Wr, b) in params:
            C = h.shape[0]
            agg = _AGG[C](h, src, dst)
            h = _TCL[C](h, agg, cnt, Wl, Wr, b.reshape(1, D_H))
        return _TCF(h, Wlin, blin.reshape(1, D_H))

    za = tower(z1)
    zb = tower(z2)
    return _DEC(za, zb, ea, eb)[:E_EDGES]
